# Initial kernel scaffold; baseline (speedup 1.0000x reference)
import jax
import jax.numpy as jnp
from jax.experimental import pallas as pl


def kernel(x, edge_index, edge_attr, batch, W_enc, b_enc, W_e, b_e, W_msg0, b_msg0, W_upd0, b_upd0, W_msg1, b_msg1, W_upd1, b_upd1, W_msg2, b_msg2, W_upd2, b_upd2, W_p1, b_p1, W_p2, b_p2):
    raise NotImplementedError("write your pallas kernel here")


# speedup vs baseline: 2.9791x; 2.9791x over previous
import functools

import jax
import jax.numpy as jnp
from jax import lax
from jax.experimental import pallas as pl
from jax.experimental.pallas import tpu as pltpu
from jax.experimental.pallas import tpu_sc as plsc

N = 10000
E = 320000
D = 128
DE = 16
H = 128
G = 256

NC = 2
NS = 16
NW = NC * NS
EB = 80
LN = 16


def _sc_layer(hW, eWb, src, dst):
    EP = E // NW
    NCHUNK = EP // EB
    RT = N // NS
    RZ = 125
    assert EP % EB == 0 and RT % RZ == 0

    mesh = plsc.VectorSubcoreMesh(core_axis_name="c", subcore_axis_name="s")

    @functools.partial(
        pl.kernel,
        mesh=mesh,
        out_type=jax.ShapeDtypeStruct((NC, NS, N // NS, H), jnp.float32),
        scratch_types=[
            pltpu.VMEM((EB,), jnp.int32),
            pltpu.VMEM((EB,), jnp.int32),
            pltpu.VMEM((EB, H), jnp.float32),
            pltpu.VMEM((EB, H), jnp.float32),
            pltpu.VMEM((RZ, H), jnp.float32),
            pltpu.VMEM_SHARED((N, H), jnp.float32),
            pltpu.SemaphoreType.DMA,
            pltpu.SemaphoreType.DMA,
        ],
    )
    def k(hW_hbm, eW_hbm, src_hbm, dst_hbm, out_hbm,
          sidx, didx, grows, erows, zbuf, acc, sem1, sem2):
        c = lax.axis_index("c")
        s = lax.axis_index("s")
        wid = s * NC + c

        zv = jnp.zeros((LN,), jnp.float32)

        def zfill(r, _):
            for cc in range(H // LN):
                zbuf[r, pl.ds(cc * LN, LN)] = zv
            return 0

        lax.fori_loop(0, RZ, zfill, 0)
        for j in range(RT // RZ):
            pltpu.sync_copy(zbuf, acc.at[pl.ds(s * RT + j * RZ, RZ)])
        plsc.subcore_barrier()

        def chunk(g, _):
            base = wid * EP + g * EB
            pltpu.sync_copy(src_hbm.at[pl.ds(base, EB)], sidx)
            pltpu.sync_copy(dst_hbm.at[pl.ds(base, EB)], didx)
            cp1 = pltpu.async_copy(hW_hbm.at[sidx], grows, sem1)
            cp2 = pltpu.async_copy(eW_hbm.at[pl.ds(base, EB), :], erows, sem2)
            cp1.wait()
            cp2.wait()

            def fuse(r, _):
                for cc in range(H // LN):
                    o = cc * LN
                    v = grows[r, pl.ds(o, LN)] + erows[r, pl.ds(o, LN)]
                    grows[r, pl.ds(o, LN)] = jnp.maximum(v, 0.0)
                return 0

            lax.fori_loop(0, EB, fuse, 0)
            pltpu.sync_copy(grows, acc.at[didx], add=True)
            return 0

        lax.fori_loop(0, NCHUNK, chunk, 0)
        plsc.subcore_barrier()

        pltpu.sync_copy(acc.at[pl.ds(s * RT, RT)], out_hbm.at[c, s])

    return k(hW, eWb, src, dst)


_NB = 1000


def _full(shape):
    return pl.BlockSpec(shape, lambda i: tuple(0 for _ in shape))


def _enc_body(x_ref, we_ref, be_ref, wm_ref, h_ref, hw_ref):
    h = jnp.dot(x_ref[...], we_ref[...], preferred_element_type=jnp.float32)
    h = jnp.maximum(h + be_ref[...], 0.0)
    h_ref[...] = h
    hw_ref[...] = jnp.dot(h, wm_ref[...], preferred_element_type=jnp.float32)


def _encoder(x, W_enc, b_enc, Wm0):
    return pl.pallas_call(
        _enc_body,
        grid=(N // _NB,),
        in_specs=[
            pl.BlockSpec((_NB, D), lambda i: (i, 0)),
            _full((D, H)),
            _full((1, H)),
            _full((H, H)),
        ],
        out_specs=[
            pl.BlockSpec((_NB, H), lambda i: (i, 0)),
            pl.BlockSpec((_NB, H), lambda i: (i, 0)),
        ],
        out_shape=[
            jax.ShapeDtypeStruct((N, H), jnp.float32),
            jax.ShapeDtypeStruct((N, H), jnp.float32),
        ],
    )(x, W_enc, b_enc, Wm0)


_EBLK = 4000


def _ew_body(ea_ref, we_ref, be_ref, wm_ref, bm_ref, o0, o1, o2):
    e = jnp.dot(ea_ref[...], we_ref[...], preferred_element_type=jnp.float32)
    e = jnp.maximum(e + be_ref[...], 0.0)
    z = jnp.dot(e, wm_ref[...], preferred_element_type=jnp.float32)
    z = z + bm_ref[...]
    o0[...] = z[:, :H]
    o1[...] = z[:, H:2 * H]
    o2[...] = z[:, 2 * H:]


def _edge_premul(edge_attr, W_e, b_e, WmAll, bmAll):
    return pl.pallas_call(
        _ew_body,
        grid=(E // _EBLK,),
        in_specs=[
            pl.BlockSpec((_EBLK, DE), lambda i: (i, 0)),
            _full((DE, H)),
            _full((1, H)),
            _full((H, 3 * H)),
            _full((1, 3 * H)),
        ],
        out_specs=[pl.BlockSpec((_EBLK, H), lambda i: (i, 0))] * 3,
        out_shape=[jax.ShapeDtypeStruct((E, H), jnp.float32)] * 3,
    )(edge_attr, W_e, b_e, WmAll, bmAll)


def _upd_body_next(h_ref, p0_ref, p1_ref, wut, wub, bu, wmn, h_out, hw_out):
    agg = p0_ref[...] + p1_ref[...]
    hn = (jnp.dot(h_ref[...], wut[...], preferred_element_type=jnp.float32)
          + jnp.dot(agg, wub[...], preferred_element_type=jnp.float32)
          + bu[...])
    hn = jnp.maximum(hn, 0.0)
    h_out[...] = hn
    hw_out[...] = jnp.dot(hn, wmn[...], preferred_element_type=jnp.float32)


def _upd_body_last(h_ref, p0_ref, p1_ref, wut, wub, bu, h_out):
    agg = p0_ref[...] + p1_ref[...]
    hn = (jnp.dot(h_ref[...], wut[...], preferred_element_type=jnp.float32)
          + jnp.dot(agg, wub[...], preferred_element_type=jnp.float32)
          + bu[...])
    h_out[...] = jnp.maximum(hn, 0.0)


def _update(h, p0, p1, Wu_t, Wu_b, bu, Wm_next):
    blk = pl.BlockSpec((_NB, H), lambda i: (i, 0))
    if Wm_next is not None:
        return pl.pallas_call(
            _upd_body_next,
            grid=(N // _NB,),
            in_specs=[blk, blk, blk, _full((H, H)), _full((H, H)),
                      _full((1, H)), _full((H, H))],
            out_specs=[blk, blk],
            out_shape=[jax.ShapeDtypeStruct((N, H), jnp.float32)] * 2,
        )(h, p0, p1, Wu_t, Wu_b, bu, Wm_next)
    return pl.pallas_call(
        _upd_body_last,
        grid=(N // _NB,),
        in_specs=[blk, blk, blk, _full((H, H)), _full((H, H)), _full((1, H))],
        out_specs=blk,
        out_shape=jax.ShapeDtypeStruct((N, H), jnp.float32),
    )(h, p0, p1, Wu_t, Wu_b, bu)


def _pool_body(b_ref, h_ref, wp1, bp1, wp2, bp2, out_ref, sums, cnts):
    i = pl.program_id(0)

    @pl.when(i == 0)
    def _():
        sums[...] = jnp.zeros_like(sums)
        cnts[...] = jnp.zeros_like(cnts)

    ids = b_ref[0, 0, :]
    gids = lax.broadcasted_iota(jnp.int32, (G, _NB), 0)
    onehot = (ids[None, :] == gids).astype(jnp.float32)
    sums[...] += jnp.dot(onehot, h_ref[...], preferred_element_type=jnp.float32)
    cnts[...] += jnp.sum(onehot, axis=1, keepdims=True)

    @pl.when(i == pl.num_programs(0) - 1)
    def _():
        gf = sums[...] / jnp.maximum(cnts[...], 1.0)
        hp = jnp.dot(gf, wp1[...], preferred_element_type=jnp.float32)
        hp = jnp.maximum(hp + bp1[...], 0.0)
        out_ref[...] = jnp.dot(hp, wp2[...],
                               preferred_element_type=jnp.float32) + bp2[...]


def _pool_head(batch2d, h, W_p1, b_p1, W_p2, b_p2):
    return pl.pallas_call(
        _pool_body,
        grid=(N // _NB,),
        in_specs=[
            pl.BlockSpec((1, 1, _NB), lambda i: (i, 0, 0)),
            pl.BlockSpec((_NB, H), lambda i: (i, 0)),
            _full((H, H)),
            _full((1, H)),
            _full((H, 1)),
            _full((1, 1)),
        ],
        out_specs=_full((G, 1)),
        out_shape=jax.ShapeDtypeStruct((G, 1), jnp.float32),
        scratch_shapes=[
            pltpu.VMEM((G, H), jnp.float32),
            pltpu.VMEM((G, 1), jnp.float32),
        ],
        compiler_params=pltpu.CompilerParams(
            dimension_semantics=("arbitrary",)),
    )(batch2d, h, W_p1, b_p1, W_p2, b_p2)


def kernel(x, edge_index, edge_attr, batch, W_enc, b_enc, W_e, b_e,
           W_msg0, b_msg0, W_upd0, b_upd0,
           W_msg1, b_msg1, W_upd1, b_upd1,
           W_msg2, b_msg2, W_upd2, b_upd2,
           W_p1, b_p1, W_p2, b_p2):
    src = edge_index[0]
    dst = edge_index[1]
    Wm = [W_msg0, W_msg1, W_msg2]
    Wu = [W_upd0, W_upd1, W_upd2]
    bu = [b_upd0, b_upd1, b_upd2]

    h, hW = _encoder(x, W_enc, b_enc.reshape(1, H), W_msg0)

    WmAll = jnp.concatenate(Wm, axis=1)
    bmAll = jnp.concatenate([b_msg0, b_msg1, b_msg2]).reshape(1, 3 * H)
    eWs = _edge_premul(edge_attr, W_e, b_e.reshape(1, H), WmAll, bmAll)

    for l in range(3):
        parts = _sc_layer(hW, eWs[l], src, dst)
        p0 = parts[0].reshape(N, H)
        p1 = parts[1].reshape(N, H)
        nxt = Wm[l + 1] if l < 2 else None
        out = _update(h, p0, p1, Wu[l][:H], Wu[l][H:],
                      bu[l].reshape(1, H), nxt)
        if l < 2:
            h, hW = out
        else:
            h = out

    return _pool_head(batch.reshape(N // _NB, 1, _NB), h, W_p1,
                      b_p1.reshape(1, H),
                      W_p2, b_p2.reshape(1, 1))
